# baseline (speedup 1.0000x reference)
import functools

import jax
import jax.numpy as jnp
from jax import lax
from jax.experimental import pallas as pl
from jax.experimental.pallas import tpu as pltpu
from jax.experimental.pallas import tpu_sc as plsc

_NC = 2
_NS = 16
_NW = _NC * _NS
_CH = 128
_ZR = 64


def _sc_sum_agg(D, n_acc, cpw, nb):
    mesh = plsc.VectorSubcoreMesh(core_axis_name="c", subcore_axis_name="s",
                                  num_cores=_NC, num_subcores=_NS)
    out_type = [jax.ShapeDtypeStruct((_NC, n_acc, D), jnp.float32)]
    scratch = [
        pltpu.VMEM((cpw, _CH), jnp.int32),
        pltpu.VMEM((cpw, _CH), jnp.int32),
        pltpu.VMEM((2, nb, _CH, D), jnp.float32),
        pltpu.VMEM((_ZR, D), jnp.float32),
        pltpu.SemaphoreType.DMA,
        pltpu.SemaphoreType.DMA,
        pltpu.SemaphoreType.DMA,
        pltpu.VMEM_SHARED((n_acc, D), jnp.float32),
    ]

    def body(h_hbm, src_hbm, dst_hbm, sum_hbm, src_v, dst_v, rows_v, zero_v,
             gsem, ssem, psem, acc_sh):
        cid = lax.axis_index("c")
        sid = lax.axis_index("s")
        wid = sid * _NC + cid

        cp_src = pltpu.async_copy(src_hbm.at[pl.ds(wid * cpw, cpw)], src_v,
                                  psem)
        cp_dst = pltpu.async_copy(dst_hbm.at[pl.ds(wid * cpw, cpw)], dst_v,
                                  psem)

        zv = jnp.zeros((16,), jnp.float32)

        def zfill(i, _):
            for k in range(D // 16):
                zero_v[i, pl.ds(k * 16, 16)] = zv
            return 0

        lax.fori_loop(0, _ZR, zfill, 0)

        rps = n_acc // _NS
        nz = rps // _ZR
        for t in range(nz):
            base = sid * rps + t * _ZR
            pltpu.async_copy(zero_v, acc_sh.at[pl.ds(base, _ZR)], ssem)
        for t in range(nz):
            pltpu.make_async_copy(
                zero_v, acc_sh.at[pl.ds(sid * rps, _ZR)], ssem).wait()
        cp_src.wait()
        cp_dst.wait()
        plsc.subcore_barrier()

        ngrp = cpw // nb

        def fire_gathers(g, h):
            for k in range(nb):
                pltpu.async_copy(h_hbm.at[src_v.at[g * nb + k]],
                                 rows_v.at[h, k], gsem)

        fire_gathers(0, 0)

        def one_group(g, h):
            for k in range(nb):
                pltpu.make_async_copy(
                    h_hbm.at[src_v.at[0]], rows_v.at[0, k], gsem).wait()

            @pl.when(g > 0)
            def _drain():
                for k in range(nb):
                    pltpu.make_async_copy(
                        rows_v.at[0, k], acc_sh.at[dst_v.at[0]],
                        ssem).wait()

            @pl.when(g + 1 < ngrp)
            def _next():
                for k in range(nb):
                    pltpu.async_copy(
                        h_hbm.at[src_v.at[(g + 1) * nb + k]],
                        rows_v.at[1 - h, k], gsem)

            for k in range(nb):
                pltpu.async_copy(rows_v.at[h, k],
                                 acc_sh.at[dst_v.at[g * nb + k]], ssem,
                                 add=True)

        def group_pair(p, _):
            one_group(p * 2, 0)
            one_group(p * 2 + 1, 1)
            return 0

        lax.fori_loop(0, ngrp // 2, group_pair, 0)
        for k in range(nb):
            pltpu.make_async_copy(
                rows_v.at[0, k], acc_sh.at[dst_v.at[0]], ssem).wait()
        plsc.subcore_barrier()

        out_base = sid * rps
        pltpu.sync_copy(acc_sh.at[pl.ds(out_base, rps)],
                        sum_hbm.at[cid, pl.ds(out_base, rps)])

    return pl.kernel(body, out_type=out_type, mesh=mesh,
                     scratch_types=scratch,
                     compiler_params=pltpu.CompilerParams(
                         use_tc_tiling_on_sc=False))


def kernel(x, edge_index, c1_W1, c1_b1, c1_W2, c1_b2,
           c2_W1, c2_b1, c2_W2, c2_b2, l1_W, l1_b, l2_W, l2_b):
    N, D = x.shape
    E = edge_index.shape[1]
    H1 = c1_W1.shape[1]
    H1p = H1 + 16
    H2 = c2_W1.shape[1]

    cpw = (-(-E // (_CH * _NW)) + 7) // 8 * 8
    n_chunks = cpw * _NW
    e_pad = n_chunks * _CH
    n_acc = (N // (_NS * _ZR) + 1) * (_NS * _ZR)
    scrap = n_acc - N

    src = edge_index[0]
    dst = edge_index[1]
    pad = e_pad - E
    src_p = jnp.concatenate([src, jnp.zeros((pad,), jnp.int32)])
    dst_p = jnp.concatenate(
        [dst, N + (jnp.arange(pad, dtype=jnp.int32) % scrap)])
    src2 = src_p.reshape(n_chunks, _CH)
    dst2 = dst_p.reshape(n_chunks, _CH)

    f32 = jnp.float32

    def tc1(x_ref, w_ref, b_ref, o_ref):
        r = jnp.maximum(
            jnp.dot(x_ref[...], w_ref[...], preferred_element_type=f32)
            + b_ref[...], 0.0)
        o_ref[...] = jnp.concatenate(
            [r, jnp.ones((N, 16), f32)], axis=1)

    h1 = pl.pallas_call(
        tc1, out_shape=jax.ShapeDtypeStruct((N, H1p), f32),
    )(x, c1_W1, c1_b1.reshape(1, H1))

    (sum1,) = _sc_sum_agg(H1p, n_acc, cpw, 2)(h1, src2, dst2)

    def tc2(x_ref, s_ref, w2_ref, b2_ref, w3_ref, b3_ref,
            x2_ref, h2_ref, inv_ref):
        s = s_ref[0][:N] + s_ref[1][:N]
        inv = 1.0 / jnp.maximum(s[:, H1:H1 + 1], 1.0)
        m = s[:, :H1] * inv
        a = (jnp.dot(x_ref[...], w2_ref[:D], preferred_element_type=f32)
             + jnp.dot(m, w2_ref[D:], preferred_element_type=f32)
             + b2_ref[...])
        x2 = jnp.maximum(a, 0.0)
        x2_ref[...] = x2
        h2_ref[...] = jnp.maximum(
            jnp.dot(x2, w3_ref[...], preferred_element_type=f32)
            + b3_ref[...], 0.0)
        inv_ref[...] = inv

    x2, h2, inv1 = pl.pallas_call(
        tc2, out_shape=[jax.ShapeDtypeStruct((N, D), f32),
                        jax.ShapeDtypeStruct((N, H2), f32),
                        jax.ShapeDtypeStruct((N, 1), f32)],
    )(x, sum1, c1_W2, c1_b2.reshape(1, -1), c2_W1, c2_b1.reshape(1, H2))

    (sum2,) = _sc_sum_agg(H2, n_acc, cpw, 4)(h2, src2, dst2)

    def tc3(x2_ref, s_ref, inv_ref, w2_ref, b2_ref, wl1_ref, bl1_ref,
            wl2_ref, bl2_ref, y_ref):
        s = s_ref[0][:N] + s_ref[1][:N]
        m = s * inv_ref[...]
        a = (jnp.dot(x2_ref[...], w2_ref[:D], preferred_element_type=f32)
             + jnp.dot(m, w2_ref[D:], preferred_element_type=f32)
             + b2_ref[...])
        o2 = jnp.maximum(a, 0.0)
        h3 = jnp.maximum(
            jnp.dot(o2, wl1_ref[...], preferred_element_type=f32)
            + bl1_ref[...], 0.0)
        y_ref[...] = (jnp.dot(h3, wl2_ref[...], preferred_element_type=f32)
                      + bl2_ref[...])

    y = pl.pallas_call(
        tc3, out_shape=jax.ShapeDtypeStruct((N, 1), f32),
    )(x2, sum2, inv1, c2_W2, c2_b2.reshape(1, -1), l1_W,
      l1_b.reshape(1, -1), l2_W, l2_b.reshape(1, -1))
    return y


# speedup vs baseline: 1.2188x; 1.2188x over previous
import functools

import jax
import jax.numpy as jnp
from jax import lax
from jax.experimental import pallas as pl
from jax.experimental.pallas import tpu as pltpu
from jax.experimental.pallas import tpu_sc as plsc

_NC = 2
_NS = 16
_NW = _NC * _NS
_CH = 128
_ZR = 64


def _sc_mean_agg(D, n_acc, cpw, with_counts, nb):
    mesh = plsc.VectorSubcoreMesh(core_axis_name="c", subcore_axis_name="s",
                                  num_cores=_NC, num_subcores=_NS)
    out_type = [jax.ShapeDtypeStruct((_NC, n_acc, D), jnp.float32)]
    scratch = [
        pltpu.VMEM((cpw, _CH), jnp.int32),
        pltpu.VMEM((cpw, _CH), jnp.int32),
        pltpu.VMEM((2, nb, _CH, D), jnp.float32),
        pltpu.VMEM((_ZR, D), jnp.float32),
        pltpu.SemaphoreType.DMA,
        pltpu.SemaphoreType.DMA,
        pltpu.SemaphoreType.DMA,
        pltpu.VMEM_SHARED((n_acc, D), jnp.float32),
    ]
    if with_counts:
        out_type.append(jax.ShapeDtypeStruct((_NC, n_acc, 16), jnp.float32))
        scratch += [
            pltpu.VMEM((_CH, 16), jnp.float32),
            pltpu.VMEM((_ZR, 16), jnp.float32),
            pltpu.SemaphoreType.DMA,
            pltpu.VMEM_SHARED((n_acc, 16), jnp.float32),
        ]

    def body(h_hbm, src_hbm, dst_hbm, *refs):
        if with_counts:
            (sum_hbm, cnt_hbm, src_v, dst_v, rows_v, zero_v, gsem, ssem,
             psem, acc_sh, ones_v, zero16_v, osem, cnt_sh) = refs
        else:
            (sum_hbm, src_v, dst_v, rows_v, zero_v, gsem, ssem, psem,
             acc_sh) = refs
        cid = lax.axis_index("c")
        sid = lax.axis_index("s")
        wid = sid * _NC + cid

        cp_src = pltpu.async_copy(src_hbm.at[pl.ds(wid * cpw, cpw)], src_v,
                                  psem)
        cp_dst = pltpu.async_copy(dst_hbm.at[pl.ds(wid * cpw, cpw)], dst_v,
                                  psem)

        zv = jnp.zeros((16,), jnp.float32)

        def zfill(i, _):
            for k in range(D // 16):
                zero_v[i, pl.ds(k * 16, 16)] = zv
            if with_counts:
                zero16_v[i, :] = zv
            return 0

        lax.fori_loop(0, _ZR, zfill, 0)
        if with_counts:
            ov = jnp.ones((16,), jnp.float32)

            def ofill(i, _):
                ones_v[i, :] = ov
                return 0

            lax.fori_loop(0, _CH, ofill, 0)

        rps = n_acc // _NS
        nz = rps // _ZR
        for t in range(nz):
            base = sid * rps + t * _ZR
            pltpu.async_copy(zero_v, acc_sh.at[pl.ds(base, _ZR)], ssem)
            if with_counts:
                pltpu.async_copy(zero16_v, cnt_sh.at[pl.ds(base, _ZR)],
                                 osem)
        for t in range(nz):
            pltpu.make_async_copy(
                zero_v, acc_sh.at[pl.ds(sid * rps, _ZR)], ssem).wait()
            if with_counts:
                pltpu.make_async_copy(
                    zero16_v, cnt_sh.at[pl.ds(sid * rps, _ZR)], osem).wait()
        cp_src.wait()
        cp_dst.wait()
        plsc.subcore_barrier()

        ngrp = cpw // nb

        def fire_gathers(g, h):
            for k in range(nb):
                pltpu.async_copy(h_hbm.at[src_v.at[g * nb + k]],
                                 rows_v.at[h, k], gsem)

        fire_gathers(0, 0)

        def one_group(g, h):
            for k in range(nb):
                pltpu.make_async_copy(
                    h_hbm.at[src_v.at[0]], rows_v.at[0, k], gsem).wait()

            @pl.when(g > 0)
            def _drain():
                for k in range(nb):
                    pltpu.make_async_copy(
                        rows_v.at[0, k], acc_sh.at[dst_v.at[0]],
                        ssem).wait()
                    if with_counts:
                        pltpu.make_async_copy(
                            ones_v, cnt_sh.at[dst_v.at[0]], osem).wait()

            @pl.when(g + 1 < ngrp)
            def _next():
                for k in range(nb):
                    pltpu.async_copy(
                        h_hbm.at[src_v.at[(g + 1) * nb + k]],
                        rows_v.at[1 - h, k], gsem)

            for k in range(nb):
                pltpu.async_copy(rows_v.at[h, k],
                                 acc_sh.at[dst_v.at[g * nb + k]], ssem,
                                 add=True)
                if with_counts:
                    pltpu.async_copy(ones_v,
                                     cnt_sh.at[dst_v.at[g * nb + k]], osem,
                                     add=True)

        def group_pair(p, _):
            one_group(p * 2, 0)
            one_group(p * 2 + 1, 1)
            return 0

        lax.fori_loop(0, ngrp // 2, group_pair, 0)
        for k in range(nb):
            pltpu.make_async_copy(
                rows_v.at[0, k], acc_sh.at[dst_v.at[0]], ssem).wait()
            if with_counts:
                pltpu.make_async_copy(
                    ones_v, cnt_sh.at[dst_v.at[0]], osem).wait()
        plsc.subcore_barrier()

        out_base = sid * rps
        pltpu.sync_copy(acc_sh.at[pl.ds(out_base, rps)],
                        sum_hbm.at[cid, pl.ds(out_base, rps)])
        if with_counts:
            pltpu.sync_copy(cnt_sh.at[pl.ds(out_base, rps)],
                            cnt_hbm.at[cid, pl.ds(out_base, rps)])

    return pl.kernel(body, out_type=out_type, mesh=mesh,
                     scratch_types=scratch,
                     compiler_params=pltpu.CompilerParams(
                         use_tc_tiling_on_sc=False))


def kernel(x, edge_index, c1_W1, c1_b1, c1_W2, c1_b2,
           c2_W1, c2_b1, c2_W2, c2_b2, l1_W, l1_b, l2_W, l2_b):
    N, D = x.shape
    E = edge_index.shape[1]
    H1 = c1_W1.shape[1]
    H2 = c2_W1.shape[1]

    cpw = (-(-E // (_CH * _NW)) + 7) // 8 * 8
    n_chunks = cpw * _NW
    e_pad = n_chunks * _CH
    n_acc = (N // (_NS * _ZR) + 1) * (_NS * _ZR)
    scrap = n_acc - N

    src = edge_index[0]
    dst = edge_index[1]
    pad = e_pad - E
    smod = scrap if scrap % 2 == 1 else scrap - 1
    src_p = jnp.concatenate([src, jnp.zeros((pad,), jnp.int32)])
    dst_p = jnp.concatenate(
        [dst, N + (jnp.arange(pad, dtype=jnp.int32) % smod)])
    src2 = src_p.reshape(cpw * _CH, _NW).T.reshape(n_chunks, _CH)
    dst2 = dst_p.reshape(cpw * _CH, _NW).T.reshape(n_chunks, _CH)

    f32 = jnp.float32

    def tc1(x_ref, w_ref, b_ref, o_ref):
        o_ref[...] = jnp.maximum(
            jnp.dot(x_ref[...], w_ref[...], preferred_element_type=f32)
            + b_ref[...], 0.0)

    h1 = pl.pallas_call(
        tc1, out_shape=jax.ShapeDtypeStruct((N, H1), f32),
    )(x, c1_W1, c1_b1.reshape(1, H1))

    sum1, cnt1 = _sc_mean_agg(H1, n_acc, cpw, True, 2)(h1, src2, dst2)

    def tc2(x_ref, s_ref, c_ref, w2_ref, b2_ref, w3_ref, b3_ref,
            x2_ref, h2_ref):
        s = s_ref[0][:N] + s_ref[1][:N]
        cnt = jnp.max(c_ref[0][:N] + c_ref[1][:N], axis=1, keepdims=True)
        m = s / jnp.maximum(cnt, 1.0)
        a = (jnp.dot(x_ref[...], w2_ref[:D], preferred_element_type=f32)
             + jnp.dot(m, w2_ref[D:], preferred_element_type=f32)
             + b2_ref[...])
        x2 = jnp.maximum(a, 0.0)
        x2_ref[...] = x2
        h2_ref[...] = jnp.maximum(
            jnp.dot(x2, w3_ref[...], preferred_element_type=f32)
            + b3_ref[...], 0.0)

    x2, h2 = pl.pallas_call(
        tc2, out_shape=[jax.ShapeDtypeStruct((N, D), f32),
                        jax.ShapeDtypeStruct((N, H2), f32)],
    )(x, sum1, cnt1, c1_W2, c1_b2.reshape(1, -1), c2_W1,
      c2_b1.reshape(1, H2))

    (sum2,) = _sc_mean_agg(H2, n_acc, cpw, False, 4)(h2, src2, dst2)

    def tc3(x2_ref, s_ref, c_ref, w2_ref, b2_ref, wl1_ref, bl1_ref,
            wl2_ref, bl2_ref, y_ref):
        s = s_ref[0][:N] + s_ref[1][:N]
        cnt = jnp.max(c_ref[0][:N] + c_ref[1][:N], axis=1, keepdims=True)
        m = s / jnp.maximum(cnt, 1.0)
        a = (jnp.dot(x2_ref[...], w2_ref[:D], preferred_element_type=f32)
             + jnp.dot(m, w2_ref[D:], preferred_element_type=f32)
             + b2_ref[...])
        o2 = jnp.maximum(a, 0.0)
        h3 = jnp.maximum(
            jnp.dot(o2, wl1_ref[...], preferred_element_type=f32)
            + bl1_ref[...], 0.0)
        y_ref[...] = (jnp.dot(h3, wl2_ref[...], preferred_element_type=f32)
                      + bl2_ref[...])

    y = pl.pallas_call(
        tc3, out_shape=jax.ShapeDtypeStruct((N, 1), f32),
    )(x2, sum2, cnt1, c2_W2, c2_b2.reshape(1, -1), l1_W,
      l1_b.reshape(1, -1), l2_W, l2_b.reshape(1, -1))
    return y


# speedup vs baseline: 1.2918x; 1.0598x over previous
import functools

import jax
import jax.numpy as jnp
from jax import lax
from jax.experimental import pallas as pl
from jax.experimental.pallas import tpu as pltpu
from jax.experimental.pallas import tpu_sc as plsc

_NC = 2
_NS = 16
_NW = _NC * _NS
_CH = 128
_ZR = 64


def _sc_mean_agg(D, n_acc, cpw, with_counts, nb):
    mesh = plsc.VectorSubcoreMesh(core_axis_name="c", subcore_axis_name="s",
                                  num_cores=_NC, num_subcores=_NS)
    out_type = [jax.ShapeDtypeStruct((_NC, n_acc, D), jnp.float32)]
    scratch = [
        pltpu.VMEM((cpw, _CH), jnp.int32),
        pltpu.VMEM((cpw, _CH), jnp.int32),
        pltpu.VMEM((2, nb, _CH, D), jnp.float32),
        pltpu.VMEM((_ZR, D), jnp.float32),
        pltpu.SemaphoreType.DMA,
        pltpu.SemaphoreType.DMA,
        pltpu.SemaphoreType.DMA,
        pltpu.VMEM_SHARED((n_acc, D), jnp.float32),
    ]
    if with_counts:
        out_type.append(jax.ShapeDtypeStruct((_NC, n_acc, 16), jnp.float32))
        scratch += [
            pltpu.VMEM((_CH, 16), jnp.float32),
            pltpu.VMEM((_ZR, 16), jnp.float32),
            pltpu.SemaphoreType.DMA,
            pltpu.VMEM_SHARED((n_acc, 16), jnp.float32),
        ]

    def body(h_hbm, src_hbm, dst_hbm, *refs):
        if with_counts:
            (sum_hbm, cnt_hbm, src_v, dst_v, rows_v, zero_v, gsem, ssem,
             psem, acc_sh, ones_v, zero16_v, osem, cnt_sh) = refs
        else:
            (sum_hbm, src_v, dst_v, rows_v, zero_v, gsem, ssem, psem,
             acc_sh) = refs
        cid = lax.axis_index("c")
        sid = lax.axis_index("s")
        wid = sid * _NC + cid

        cp_src = pltpu.async_copy(src_hbm.at[pl.ds(wid * cpw, cpw)], src_v,
                                  psem)
        cp_dst = pltpu.async_copy(dst_hbm.at[pl.ds(wid * cpw, cpw)], dst_v,
                                  psem)

        zv = jnp.zeros((16,), jnp.float32)

        def zfill(i, _):
            for k in range(D // 16):
                zero_v[i, pl.ds(k * 16, 16)] = zv
            if with_counts:
                zero16_v[i, :] = zv
            return 0

        lax.fori_loop(0, _ZR, zfill, 0)
        if with_counts:
            ov = jnp.ones((16,), jnp.float32)

            def ofill(i, _):
                ones_v[i, :] = ov
                return 0

            lax.fori_loop(0, _CH, ofill, 0)

        rps = n_acc // _NS
        nz = rps // _ZR
        for t in range(nz):
            base = sid * rps + t * _ZR
            pltpu.async_copy(zero_v, acc_sh.at[pl.ds(base, _ZR)], ssem)
            if with_counts:
                pltpu.async_copy(zero16_v, cnt_sh.at[pl.ds(base, _ZR)],
                                 osem)
        for t in range(nz):
            pltpu.make_async_copy(
                zero_v, acc_sh.at[pl.ds(sid * rps, _ZR)], ssem).wait()
            if with_counts:
                pltpu.make_async_copy(
                    zero16_v, cnt_sh.at[pl.ds(sid * rps, _ZR)], osem).wait()
        cp_src.wait()
        cp_dst.wait()
        plsc.subcore_barrier()

        ngrp = cpw // nb

        def fire_gathers(g, h):
            for k in range(nb):
                pltpu.async_copy(h_hbm.at[src_v.at[g * nb + k]],
                                 rows_v.at[h, k], gsem)

        fire_gathers(0, 0)

        def one_group(g, h):
            for k in range(nb):
                pltpu.make_async_copy(
                    h_hbm.at[src_v.at[0]], rows_v.at[0, k], gsem).wait()

            @pl.when(g > 0)
            def _drain():
                for k in range(nb):
                    pltpu.make_async_copy(
                        rows_v.at[0, k], acc_sh.at[dst_v.at[0]],
                        ssem).wait()
                    if with_counts:
                        pltpu.make_async_copy(
                            ones_v, cnt_sh.at[dst_v.at[0]], osem).wait()

            @pl.when(g + 1 < ngrp)
            def _next():
                for k in range(nb):
                    pltpu.async_copy(
                        h_hbm.at[src_v.at[(g + 1) * nb + k]],
                        rows_v.at[1 - h, k], gsem)

            for k in range(nb):
                pltpu.async_copy(rows_v.at[h, k],
                                 acc_sh.at[dst_v.at[g * nb + k]], ssem,
                                 add=True)
                if with_counts:
                    pltpu.async_copy(ones_v,
                                     cnt_sh.at[dst_v.at[g * nb + k]], osem,
                                     add=True)

        def group_pair(p, _):
            one_group(p * 2, 0)
            one_group(p * 2 + 1, 1)
            return 0

        lax.fori_loop(0, ngrp // 2, group_pair, 0)
        for k in range(nb):
            pltpu.make_async_copy(
                rows_v.at[0, k], acc_sh.at[dst_v.at[0]], ssem).wait()
            if with_counts:
                pltpu.make_async_copy(
                    ones_v, cnt_sh.at[dst_v.at[0]], osem).wait()
        plsc.subcore_barrier()

        out_base = sid * rps
        pltpu.sync_copy(acc_sh.at[pl.ds(out_base, rps)],
                        sum_hbm.at[cid, pl.ds(out_base, rps)])
        if with_counts:
            pltpu.sync_copy(cnt_sh.at[pl.ds(out_base, rps)],
                            cnt_hbm.at[cid, pl.ds(out_base, rps)])

    return pl.kernel(body, out_type=out_type, mesh=mesh,
                     scratch_types=scratch,
                     compiler_params=pltpu.CompilerParams(
                         use_tc_tiling_on_sc=False))


def kernel(x, edge_index, c1_W1, c1_b1, c1_W2, c1_b2,
           c2_W1, c2_b1, c2_W2, c2_b2, l1_W, l1_b, l2_W, l2_b):
    N, D = x.shape
    E = edge_index.shape[1]
    H1 = c1_W1.shape[1]
    H2 = c2_W1.shape[1]

    cpw = (-(-E // (_CH * _NW)) + 7) // 8 * 8
    n_chunks = cpw * _NW
    e_pad = n_chunks * _CH
    n_acc = (N // (_NS * _ZR) + 1) * (_NS * _ZR)
    scrap = n_acc - N

    src = edge_index[0]
    dst = edge_index[1]
    pad = e_pad - E
    src_p = jnp.concatenate([src, jnp.zeros((pad,), jnp.int32)])
    dst_p = jnp.concatenate(
        [dst, N + (jnp.arange(pad, dtype=jnp.int32) % scrap)])
    src2 = src_p.reshape(n_chunks, _CH)
    dst2 = dst_p.reshape(n_chunks, _CH)

    f32 = jnp.float32

    def tc1(x_ref, w_ref, b_ref, o_ref):
        o_ref[...] = jnp.maximum(
            jnp.dot(x_ref[...], w_ref[...], preferred_element_type=f32)
            + b_ref[...], 0.0)

    h1 = pl.pallas_call(
        tc1, out_shape=jax.ShapeDtypeStruct((N, H1), f32),
    )(x, c1_W1, c1_b1.reshape(1, H1))

    sum1, cnt1 = _sc_mean_agg(H1, n_acc, cpw, True, 2)(h1, src2, dst2)

    def tc2(x_ref, s_ref, c_ref, w2_ref, b2_ref, w3_ref, b3_ref,
            x2_ref, h2_ref):
        s = s_ref[0][:N] + s_ref[1][:N]
        cnt = jnp.max(c_ref[0][:N] + c_ref[1][:N], axis=1, keepdims=True)
        m = s / jnp.maximum(cnt, 1.0)
        a = (jnp.dot(x_ref[...], w2_ref[:D], preferred_element_type=f32)
             + jnp.dot(m, w2_ref[D:], preferred_element_type=f32)
             + b2_ref[...])
        x2 = jnp.maximum(a, 0.0)
        x2_ref[...] = x2
        h2_ref[...] = jnp.maximum(
            jnp.dot(x2, w3_ref[...], preferred_element_type=f32)
            + b3_ref[...], 0.0)

    x2, h2 = pl.pallas_call(
        tc2, out_shape=[jax.ShapeDtypeStruct((N, D), f32),
                        jax.ShapeDtypeStruct((N, H2), f32)],
    )(x, sum1, cnt1, c1_W2, c1_b2.reshape(1, -1), c2_W1,
      c2_b1.reshape(1, H2))

    (sum2,) = _sc_mean_agg(H2, n_acc, cpw, False, 4)(h2, src2, dst2)

    def tc3(x2_ref, s_ref, c_ref, w2_ref, b2_ref, wl1_ref, bl1_ref,
            wl2_ref, bl2_ref, y_ref):
        s = s_ref[0][:N] + s_ref[1][:N]
        cnt = jnp.max(c_ref[0][:N] + c_ref[1][:N], axis=1, keepdims=True)
        m = s / jnp.maximum(cnt, 1.0)
        a = (jnp.dot(x2_ref[...], w2_ref[:D], preferred_element_type=f32)
             + jnp.dot(m, w2_ref[D:], preferred_element_type=f32)
             + b2_ref[...])
        o2 = jnp.maximum(a, 0.0)
        h3 = jnp.maximum(
            jnp.dot(o2, wl1_ref[...], preferred_element_type=f32)
            + bl1_ref[...], 0.0)
        y_ref[...] = (jnp.dot(h3, wl2_ref[...], preferred_element_type=f32)
                      + bl2_ref[...])

    y = pl.pallas_call(
        tc3, out_shape=jax.ShapeDtypeStruct((N, 1), f32),
    )(x2, sum2, cnt1, c2_W2, c2_b2.reshape(1, -1), l1_W,
      l1_b.reshape(1, -1), l2_W, l2_b.reshape(1, -1))
    return y


# speedup vs baseline: 1.2971x; 1.0041x over previous
import functools

import jax
import jax.numpy as jnp
from jax import lax
from jax.experimental import pallas as pl
from jax.experimental.pallas import tpu as pltpu
from jax.experimental.pallas import tpu_sc as plsc

_NC = 2
_NS = 16
_NW = _NC * _NS
_CH = 128
_ZR = 64


def _sc_mean_agg(D, n_acc, cpw, with_counts, nb):
    mesh = plsc.VectorSubcoreMesh(core_axis_name="c", subcore_axis_name="s",
                                  num_cores=_NC, num_subcores=_NS)
    out_type = [jax.ShapeDtypeStruct((_NC, n_acc, D), jnp.float32)]
    scratch = [
        pltpu.VMEM((cpw, _CH), jnp.int32),
        pltpu.VMEM((cpw, _CH), jnp.int32),
        pltpu.VMEM((2, nb, _CH, D), jnp.float32),
        pltpu.VMEM((_ZR, D), jnp.float32),
        pltpu.SemaphoreType.DMA,
        pltpu.SemaphoreType.DMA,
        pltpu.SemaphoreType.DMA,
        pltpu.VMEM_SHARED((n_acc, D), jnp.float32),
    ]
    if with_counts:
        out_type.append(jax.ShapeDtypeStruct((_NC, n_acc, 16), jnp.float32))
        scratch += [
            pltpu.VMEM((_CH, 16), jnp.float32),
            pltpu.VMEM((_ZR, 16), jnp.float32),
            pltpu.SemaphoreType.DMA,
            pltpu.VMEM_SHARED((n_acc, 16), jnp.float32),
        ]

    def body(h_hbm, src_hbm, dst_hbm, *refs):
        if with_counts:
            (sum_hbm, cnt_hbm, src_v, dst_v, rows_v, zero_v, gsem, ssem,
             psem, acc_sh, ones_v, zero16_v, osem, cnt_sh) = refs
        else:
            (sum_hbm, src_v, dst_v, rows_v, zero_v, gsem, ssem, psem,
             acc_sh) = refs
        cid = lax.axis_index("c")
        sid = lax.axis_index("s")
        wid = sid * _NC + cid

        cp_src = pltpu.async_copy(src_hbm.at[pl.ds(wid * cpw, cpw)], src_v,
                                  psem)
        cp_dst = pltpu.async_copy(dst_hbm.at[pl.ds(wid * cpw, cpw)], dst_v,
                                  psem)

        zv = jnp.zeros((16,), jnp.float32)

        def zfill(i, _):
            for k in range(D // 16):
                zero_v[i, pl.ds(k * 16, 16)] = zv
            if with_counts:
                zero16_v[i, :] = zv
            return 0

        lax.fori_loop(0, _ZR, zfill, 0)
        if with_counts:
            ov = jnp.ones((16,), jnp.float32)

            def ofill(i, _):
                ones_v[i, :] = ov
                return 0

            lax.fori_loop(0, _CH, ofill, 0)

        rps = n_acc // _NS
        nz = rps // _ZR
        for t in range(nz):
            base = sid * rps + t * _ZR
            pltpu.async_copy(zero_v, acc_sh.at[pl.ds(base, _ZR)], ssem)
            if with_counts:
                pltpu.async_copy(zero16_v, cnt_sh.at[pl.ds(base, _ZR)],
                                 osem)
        for t in range(nz):
            pltpu.make_async_copy(
                zero_v, acc_sh.at[pl.ds(sid * rps, _ZR)], ssem).wait()
            if with_counts:
                pltpu.make_async_copy(
                    zero16_v, cnt_sh.at[pl.ds(sid * rps, _ZR)], osem).wait()
        cp_src.wait()
        cp_dst.wait()
        plsc.subcore_barrier()

        ngrp = cpw // nb

        def fire_gathers(g, h):
            for k in range(nb):
                pltpu.async_copy(h_hbm.at[src_v.at[g * nb + k]],
                                 rows_v.at[h, k], gsem)

        fire_gathers(0, 0)

        def one_group(g, h):
            for k in range(nb):
                pltpu.make_async_copy(
                    h_hbm.at[src_v.at[0]], rows_v.at[0, k], gsem).wait()

            @pl.when(g > 0)
            def _drain():
                for k in range(nb):
                    pltpu.make_async_copy(
                        rows_v.at[0, k], acc_sh.at[dst_v.at[0]],
                        ssem).wait()
                    if with_counts:
                        pltpu.make_async_copy(
                            ones_v, cnt_sh.at[dst_v.at[0]], osem).wait()

            @pl.when(g + 1 < ngrp)
            def _next():
                for k in range(nb):
                    pltpu.async_copy(
                        h_hbm.at[src_v.at[(g + 1) * nb + k]],
                        rows_v.at[1 - h, k], gsem)

            for k in range(nb):
                pltpu.async_copy(rows_v.at[h, k],
                                 acc_sh.at[dst_v.at[g * nb + k]], ssem,
                                 add=True)
                if with_counts:
                    pltpu.async_copy(ones_v,
                                     cnt_sh.at[dst_v.at[g * nb + k]], osem,
                                     add=True)

        def group_pair(p, _):
            one_group(p * 2, 0)
            one_group(p * 2 + 1, 1)
            return 0

        lax.fori_loop(0, ngrp // 2, group_pair, 0)
        for k in range(nb):
            pltpu.make_async_copy(
                rows_v.at[0, k], acc_sh.at[dst_v.at[0]], ssem).wait()
            if with_counts:
                pltpu.make_async_copy(
                    ones_v, cnt_sh.at[dst_v.at[0]], osem).wait()
        plsc.subcore_barrier()

        out_base = sid * rps
        pltpu.sync_copy(acc_sh.at[pl.ds(out_base, rps)],
                        sum_hbm.at[cid, pl.ds(out_base, rps)])
        if with_counts:
            pltpu.sync_copy(cnt_sh.at[pl.ds(out_base, rps)],
                            cnt_hbm.at[cid, pl.ds(out_base, rps)])

    return pl.kernel(body, out_type=out_type, mesh=mesh,
                     scratch_types=scratch,
                     compiler_params=pltpu.CompilerParams(
                         use_tc_tiling_on_sc=False))


def kernel(x, edge_index, c1_W1, c1_b1, c1_W2, c1_b2,
           c2_W1, c2_b1, c2_W2, c2_b2, l1_W, l1_b, l2_W, l2_b):
    N, D = x.shape
    E = edge_index.shape[1]
    H1 = c1_W1.shape[1]
    H2 = c2_W1.shape[1]

    cpw = (-(-E // (_CH * _NW)) + 7) // 8 * 8
    n_chunks = cpw * _NW
    e_pad = n_chunks * _CH
    n_acc = (N // (_NS * _ZR) + 1) * (_NS * _ZR)
    scrap = n_acc - N

    src = edge_index[0]
    dst = edge_index[1]
    pad = e_pad - E
    src_p = jnp.concatenate([src, jnp.zeros((pad,), jnp.int32)])
    dst_p = jnp.concatenate(
        [dst, N + (jnp.arange(pad, dtype=jnp.int32) % scrap)])
    src2 = src_p.reshape(n_chunks, _CH)
    dst2 = dst_p.reshape(n_chunks, _CH)

    f32 = jnp.float32

    def tc1(x_ref, w_ref, b_ref, o_ref):
        o_ref[...] = jnp.maximum(
            jnp.dot(x_ref[...], w_ref[...], preferred_element_type=f32)
            + b_ref[...], 0.0)

    h1 = pl.pallas_call(
        tc1, out_shape=jax.ShapeDtypeStruct((N, H1), f32),
    )(x, c1_W1, c1_b1.reshape(1, H1))

    sum1, cnt1 = _sc_mean_agg(H1, n_acc, cpw, True, 2)(h1, src2, dst2)

    def tc2(x_ref, s_ref, c_ref, w2_ref, b2_ref, w3_ref, b3_ref,
            x2_ref, h2_ref):
        s = s_ref[0][:N] + s_ref[1][:N]
        cnt = jnp.max(c_ref[0][:N] + c_ref[1][:N], axis=1, keepdims=True)
        m = s / jnp.maximum(cnt, 1.0)
        a = (jnp.dot(x_ref[...], w2_ref[:D], preferred_element_type=f32)
             + jnp.dot(m, w2_ref[D:], preferred_element_type=f32)
             + b2_ref[...])
        x2 = jnp.maximum(a, 0.0)
        x2_ref[...] = x2
        h2_ref[...] = jnp.maximum(
            jnp.dot(x2, w3_ref[...], preferred_element_type=f32)
            + b3_ref[...], 0.0)

    x2, h2 = pl.pallas_call(
        tc2, out_shape=[jax.ShapeDtypeStruct((N, D), f32),
                        jax.ShapeDtypeStruct((N, H2), f32)],
    )(x, sum1, cnt1, c1_W2, c1_b2.reshape(1, -1), c2_W1,
      c2_b1.reshape(1, H2))

    (sum2,) = _sc_mean_agg(H2, n_acc, cpw, False, 5)(h2, src2, dst2)

    def tc3(x2_ref, s_ref, c_ref, w2_ref, b2_ref, wl1_ref, bl1_ref,
            wl2_ref, bl2_ref, y_ref):
        s = s_ref[0][:N] + s_ref[1][:N]
        cnt = jnp.max(c_ref[0][:N] + c_ref[1][:N], axis=1, keepdims=True)
        m = s / jnp.maximum(cnt, 1.0)
        a = (jnp.dot(x2_ref[...], w2_ref[:D], preferred_element_type=f32)
             + jnp.dot(m, w2_ref[D:], preferred_element_type=f32)
             + b2_ref[...])
        o2 = jnp.maximum(a, 0.0)
        h3 = jnp.maximum(
            jnp.dot(o2, wl1_ref[...], preferred_element_type=f32)
            + bl1_ref[...], 0.0)
        y_ref[...] = (jnp.dot(h3, wl2_ref[...], preferred_element_type=f32)
                      + bl2_ref[...])

    y = pl.pallas_call(
        tc3, out_shape=jax.ShapeDtypeStruct((N, 1), f32),
    )(x2, sum2, cnt1, c2_W2, c2_b2.reshape(1, -1), l1_W,
      l1_b.reshape(1, -1), l2_W, l2_b.reshape(1, -1))
    return y
